# baseline (speedup 1.0000x reference)
import functools

import jax
import jax.numpy as jnp
from jax import lax
from jax.experimental import pallas as pl
from jax.experimental.pallas import tpu as pltpu
from jax.experimental.pallas import tpu_sc as plsc

N = 1024
E = 32768
EMB = 128
L = 4
K_DAGNN = 5
NUM_GRAPHS = 16

_NC = 2
_NS = 16
_NW = _NC * _NS
_C = 128
_EPT = E // _NW
_NCH = _EPT // _C
_RPS = N // _NS

_TI = 128
_TJ = 128


def _mesh():
    return plsc.VectorSubcoreMesh(core_axis_name="c", subcore_axis_name="s")


def _zero_fill(buf, rows, width):
    def zrow(r, _):
        def zcol(k, __):
            buf[r, pl.ds(k * 16, 16)] = jnp.zeros((16,), jnp.float32)
            return 0
        lax.fori_loop(0, width // 16, zcol, 0, unroll=True)
        return 0
    lax.fori_loop(0, rows, zrow, 0)


@functools.partial(
    pl.kernel, mesh=_mesh(),
    out_type=[jax.ShapeDtypeStruct((_NC, N, EMB), jnp.float32),
              jax.ShapeDtypeStruct((_NC, N, EMB), jnp.float32)],
    scratch_types=[
        pltpu.VMEM((_NCH, _C), jnp.int32),
        pltpu.VMEM((_NCH, _C), jnp.int32),
        pltpu.VMEM((_C, EMB), jnp.float32),
        pltpu.VMEM((_C, EMB), jnp.float32),
        pltpu.VMEM((_C, EMB), jnp.float32),
        pltpu.VMEM((_C, EMB), jnp.float32),
        pltpu.VMEM((_C, EMB), jnp.float32),
        pltpu.VMEM((16,), jnp.float32),
        pltpu.VMEM((_RPS, EMB), jnp.float32),
        pltpu.VMEM_SHARED((N, EMB), jnp.float32),
        pltpu.VMEM_SHARED((N, EMB), jnp.float32),
        pltpu.VMEM_SHARED((N, EMB), jnp.float32),
        pltpu.SemaphoreType.DMA,
        pltpu.SemaphoreType.DMA,
    ])
def _genconv_sc(hh_hbm, eattr_hbm, src2d_hbm, dst2d_hbm, t_hbm,
                outp_hbm, outm_hbm,
                sidx, didx, rowsa, rowsb, erows, prows, mrows, tv, zbuf,
                htab, accp, accm, gsa, gsb):
    c = lax.axis_index("c")
    s = lax.axis_index("s")
    wid = s * _NC + c
    _zero_fill(zbuf, _RPS, EMB)
    pltpu.sync_copy(zbuf, accp.at[pl.ds(s * _RPS, _RPS)])
    pltpu.sync_copy(zbuf, accm.at[pl.ds(s * _RPS, _RPS)])
    pltpu.sync_copy(t_hbm, tv)
    pltpu.sync_copy(hh_hbm.at[pl.ds(s * _RPS, _RPS)],
                    htab.at[pl.ds(s * _RPS, _RPS)])
    pltpu.sync_copy(src2d_hbm.at[pl.ds(wid * _NCH, _NCH)], sidx)
    pltpu.sync_copy(dst2d_hbm.at[pl.ds(wid * _NCH, _NCH)], didx)
    plsc.subcore_barrier()

    ebufs = (erows, rowsb)
    esems = (gsa, gsb)
    pltpu.async_copy(eattr_hbm.at[pl.ds(wid * _EPT, _C)], erows, gsa)
    for ci in range(_NCH):
        eb = ebufs[ci % 2]
        if ci + 1 < _NCH:
            pltpu.async_copy(
                eattr_hbm.at[pl.ds(wid * _EPT + (ci + 1) * _C, _C)],
                ebufs[(ci + 1) % 2], esems[(ci + 1) % 2])
        rows = rowsa
        pltpu.sync_copy(htab.at[sidx.at[ci]], rows)
        pltpu.make_async_copy(
            eattr_hbm.at[pl.ds(wid * _EPT + ci * _C, _C)], eb,
            esems[ci % 2]).wait()
        tvv = tv[...]

        def row(r, __):
            for k in range(EMB // 16):
                a = rows[r, pl.ds(k * 16, 16)]
                e = eb[r, pl.ds(k * 16, 16)]
                m = jnp.maximum(a + e, 0.0) + 1e-7
                p = jnp.exp(m * tvv)
                prows[r, pl.ds(k * 16, 16)] = p
                mrows[r, pl.ds(k * 16, 16)] = m * p
            return 0

        lax.fori_loop(0, _C, row, 0, unroll=4)
        pltpu.sync_copy(prows, accp.at[didx.at[ci]], add=True)
        pltpu.sync_copy(mrows, accm.at[didx.at[ci]], add=True)
    plsc.subcore_barrier()
    pltpu.sync_copy(accp.at[pl.ds(s * _RPS, _RPS)],
                    outp_hbm.at[c, pl.ds(s * _RPS, _RPS)])
    pltpu.sync_copy(accm.at[pl.ds(s * _RPS, _RPS)],
                    outm_hbm.at[c, pl.ds(s * _RPS, _RPS)])


@functools.partial(
    pl.kernel, mesh=_mesh(),
    out_type=jax.ShapeDtypeStruct((_NC, N, EMB), jnp.float32),
    scratch_types=[
        pltpu.VMEM((_NCH, _C), jnp.int32),
        pltpu.VMEM((_NCH, _C), jnp.int32),
        pltpu.VMEM((_C, EMB), jnp.float32),
        pltpu.VMEM((_C, EMB), jnp.float32),
        pltpu.VMEM((_RPS, EMB), jnp.float32),
        pltpu.VMEM_SHARED((N, EMB), jnp.float32),
        pltpu.VMEM_SHARED((N, EMB), jnp.float32),
        pltpu.SemaphoreType.DMA,
        pltpu.SemaphoreType.DMA,
    ])
def _hop_sc(z_hbm, src2d_hbm, dst2d_hbm, out_hbm, sidx, didx, rowsa, rowsb,
            zbuf, ztab, acc, gsa, gsb):
    c = lax.axis_index("c")
    s = lax.axis_index("s")
    wid = s * _NC + c
    _zero_fill(zbuf, _RPS, EMB)
    pltpu.sync_copy(zbuf, acc.at[pl.ds(s * _RPS, _RPS)])
    pltpu.sync_copy(z_hbm.at[pl.ds(s * _RPS, _RPS)],
                    ztab.at[pl.ds(s * _RPS, _RPS)])
    pltpu.sync_copy(src2d_hbm.at[pl.ds(wid * _NCH, _NCH)], sidx)
    pltpu.sync_copy(dst2d_hbm.at[pl.ds(wid * _NCH, _NCH)], didx)
    plsc.subcore_barrier()

    for ci in range(_NCH):
        pltpu.sync_copy(ztab.at[sidx.at[ci]], rowsa)
        pltpu.sync_copy(rowsa, acc.at[didx.at[ci]], add=True)
    plsc.subcore_barrier()
    pltpu.sync_copy(acc.at[pl.ds(s * _RPS, _RPS)],
                    out_hbm.at[c, pl.ds(s * _RPS, _RPS)])


def _pair_body(xi_ref, xj_ref, bi_ref, bj_ref, w_ref, fb_ref,
               mdp_ref, mask_ref, cnt_ref):
    i = pl.program_id(0)
    j = pl.program_id(1)

    @pl.when((i == 0) & (j == 0))
    def _():
        cnt_ref[...] = jnp.zeros((1, 1), jnp.float32)

    bi = bi_ref[...]
    bj = bj_ref[...]
    overlap = (jnp.min(bi) <= jnp.max(bj)) & (jnp.min(bj) <= jnp.max(bi))

    @pl.when(overlap)
    def _():
        xi = xi_ref[...]
        xj = xj_ref[...]
        m = jnp.maximum(xi[:, None, :], xj[None, :, :])
        dp = jax.lax.dot_general(
            m.reshape(_TI * _TJ, EMB), w_ref[...],
            (((1,), (0,)), ((), ())),
            preferred_element_type=jnp.float32).reshape(_TI, _TJ)
        dp = dp + fb_ref[0, 0]
        same = (bi == bj).astype(jnp.float32)
        ri = jax.lax.broadcasted_iota(jnp.int32, (_TI, _TJ), 0) + i * _TI
        ci = jax.lax.broadcasted_iota(jnp.int32, (_TI, _TJ), 1) + j * _TJ
        msk = same * (1.0 - (ri == ci).astype(jnp.float32))
        mdp_ref[...] = jax.nn.relu(dp * msk)
        mask_ref[...] = msk
        cnt_ref[...] += jnp.sum(msk).reshape(1, 1)

    @pl.when(jnp.logical_not(overlap))
    def _():
        mdp_ref[...] = jnp.zeros((_TI, _TJ), jnp.float32)
        mask_ref[...] = jnp.zeros((_TI, _TJ), jnp.float32)


def _pairwise_head(xs, batch, fc_w, fc_b):
    batchf = batch.astype(jnp.float32)
    bi = batchf.reshape(N, 1)
    bj = batchf.reshape(1, N)
    fb = fc_b.reshape(1, 1)
    grid = (N // _TI, N // _TJ)
    mdp, mask, cnt = pl.pallas_call(
        _pair_body,
        grid=grid,
        in_specs=[
            pl.BlockSpec((_TI, EMB), lambda i, j: (i, 0)),
            pl.BlockSpec((_TJ, EMB), lambda i, j: (j, 0)),
            pl.BlockSpec((_TI, 1), lambda i, j: (i, 0)),
            pl.BlockSpec((1, _TJ), lambda i, j: (0, j)),
            pl.BlockSpec((EMB, 1), lambda i, j: (0, 0)),
            pl.BlockSpec((1, 1), lambda i, j: (0, 0)),
        ],
        out_specs=[
            pl.BlockSpec((_TI, _TJ), lambda i, j: (i, j)),
            pl.BlockSpec((_TI, _TJ), lambda i, j: (i, j)),
            pl.BlockSpec((1, 1), lambda i, j: (0, 0)),
        ],
        out_shape=[
            jax.ShapeDtypeStruct((N, N), jnp.float32),
            jax.ShapeDtypeStruct((N, N), jnp.float32),
            jax.ShapeDtypeStruct((1, 1), jnp.float32),
        ],
    )(xs, xs, bi, bj, fc_w, fb)
    return mdp, mask, cnt[0, 0]


def _bn(x, g, b):
    mu = jnp.mean(x, axis=0)
    var = jnp.var(x, axis=0)
    return g * (x - mu) / jnp.sqrt(var + 1e-5) + b


def _aggr_from_parts(pp, mm):
    den = pp[0] + pp[1]
    num = mm[0] + mm[1]
    return num / (den + 1e-16)


def kernel(x, edge_index, edge_attr, batch, atom_emb, bond_emb, vn_emb,
           conv_w1, conv_b1, conv_g1, conv_be1, conv_w2, conv_b2, conv_t,
           ln_g, ln_b, vn_w1, vn_b1, vn_g1, vn_be1, vn_w2, vn_b2, vn_g2,
           vn_be2, proj_w, proj_b, fc_w, fc_b):
    src = edge_index[0]
    dst = edge_index[1]
    src2d = src.reshape(E // _C, _C)
    dst2d = dst.reshape(E // _C, _C)

    ea = edge_attr.astype(jnp.float32)
    iota8 = jnp.arange(8, dtype=jnp.float32)
    eattr = jnp.zeros((E, EMB), jnp.float32)
    for k in range(3):
        oh = (ea[:, k:k + 1] == iota8[None, :]).astype(jnp.float32)
        eattr = eattr + oh @ bond_emb[k]

    xf = x.astype(jnp.float32)
    iota64 = jnp.arange(64, dtype=jnp.float32)
    h = jnp.zeros((N, EMB), jnp.float32)
    for k in range(9):
        oh = (xf[:, k:k + 1] == iota64[None, :]).astype(jnp.float32)
        h = h + oh @ atom_emb[k]

    ohb = (batch.astype(jnp.float32)[:, None]
           == jnp.arange(NUM_GRAPHS, dtype=jnp.float32)[None, :]
           ).astype(jnp.float32)

    vn = jnp.broadcast_to(vn_emb, (NUM_GRAPHS, EMB))
    h = h + ohb @ vn

    def genconv(hh, i):
        t16 = jnp.full((16,), conv_t[i], jnp.float32)
        pp, mm = _genconv_sc(hh, eattr, src2d, dst2d, t16)
        aggr = _aggr_from_parts(pp, mm)
        out = hh + aggr
        out = jax.nn.relu(_bn(out @ conv_w1[i] + conv_b1[i],
                              conv_g1[i], conv_be1[i]))
        return out @ conv_w2[i] + conv_b2[i]

    h = genconv(h, 0)
    for i in range(1, L):
        hh = jax.nn.relu(_bn(h, ln_g[i], ln_b[i]))
        h = h + genconv(hh, i)
        j = i - 1
        vtmp = ohb.T @ h + vn
        v = jax.nn.relu(_bn(vtmp @ vn_w1[j] + vn_b1[j], vn_g1[j], vn_be1[j]))
        vn = jax.nn.relu(_bn(v @ vn_w2[j] + vn_b2[j], vn_g2[j], vn_be2[j]))
        h = h + ohb @ vn
    h = jax.nn.relu(_bn(h, ln_g[0], ln_b[0]))
    h = h + ohb @ vn

    onesm = jnp.full((N, EMB), 1.0, jnp.float32) * jnp.where(
        jnp.isnan(h[0, 0]), 2.0, 1.0)
    degp = _hop_sc(onesm, src2d, dst2d)
    deg = degp[0, :, 0] + degp[1, :, 0] + 1.0
    dis = deg ** -0.5
    xk = h
    z = dis[:, None] * xk
    preds = [xk]
    for _ in range(K_DAGNN):
        parts = _hop_sc(z, src2d, dst2d)
        xk = dis[:, None] * (parts[0] + parts[1] + z)
        z = dis[:, None] * xk
        preds.append(xk)
    scores = [jax.nn.sigmoid(p @ proj_w + proj_b[0]) for p in preds]
    xs = jnp.zeros((N, EMB), jnp.float32)
    for rs, p in zip(scores, preds):
        xs = xs + rs * p

    mdp, mask, count = _pairwise_head(xs, batch, fc_w, fc_b)
    return (mdp, mask, count)


# speedup vs baseline: 1.8850x; 1.8850x over previous
import functools

import jax
import jax.numpy as jnp
from jax import lax
from jax.experimental import pallas as pl
from jax.experimental.pallas import tpu as pltpu
from jax.experimental.pallas import tpu_sc as plsc

N = 1024
E = 32768
EMB = 128
L = 4
K_DAGNN = 5
NUM_GRAPHS = 16

_NC = 2
_NS = 16
_NW = _NC * _NS
_C = 128
_EPT = E // _NW
_NCH = _EPT // _C
_RPS = N // _NS

_TI = 128
_TJ = 128


def _mesh():
    return plsc.VectorSubcoreMesh(core_axis_name="c", subcore_axis_name="s")


def _zero_fill(buf, rows, width):
    def zrow(r, _):
        def zcol(k, __):
            buf[r, pl.ds(k * 16, 16)] = jnp.zeros((16,), jnp.float32)
            return 0
        lax.fori_loop(0, width // 16, zcol, 0, unroll=True)
        return 0
    lax.fori_loop(0, rows, zrow, 0)


@functools.partial(
    pl.kernel, mesh=_mesh(),
    out_type=[jax.ShapeDtypeStruct((_NC, N, EMB), jnp.float32),
              jax.ShapeDtypeStruct((_NC, N, EMB), jnp.float32)],
    scratch_types=[
        pltpu.VMEM((_NCH, _C), jnp.int32),
        pltpu.VMEM((_NCH, _C), jnp.int32),
        pltpu.VMEM((_C, EMB), jnp.float32),
        pltpu.VMEM((_C, EMB), jnp.float32),
        pltpu.VMEM((_C, EMB), jnp.float32),
        pltpu.VMEM((_C, EMB), jnp.float32),
        pltpu.VMEM((_C, EMB), jnp.float32),
        pltpu.VMEM((16,), jnp.float32),
        pltpu.VMEM((_RPS, EMB), jnp.float32),
        pltpu.VMEM_SHARED((N, EMB), jnp.float32),
        pltpu.VMEM_SHARED((N, EMB), jnp.float32),
        pltpu.VMEM_SHARED((N, EMB), jnp.float32),
        pltpu.SemaphoreType.DMA,
        pltpu.SemaphoreType.DMA,
    ])
def _genconv_sc(hh_hbm, eattr_hbm, src2d_hbm, dst2d_hbm, t_hbm,
                outp_hbm, outm_hbm,
                sidx, didx, rowsa, rowsb, erows, prows, mrows, tv, zbuf,
                htab, accp, accm, gsa, gsb):
    c = lax.axis_index("c")
    s = lax.axis_index("s")
    wid = s * _NC + c
    _zero_fill(zbuf, _RPS, EMB)
    pltpu.sync_copy(zbuf, accp.at[pl.ds(s * _RPS, _RPS)])
    pltpu.sync_copy(zbuf, accm.at[pl.ds(s * _RPS, _RPS)])
    pltpu.sync_copy(t_hbm, tv)
    pltpu.sync_copy(hh_hbm.at[pl.ds(s * _RPS, _RPS)],
                    htab.at[pl.ds(s * _RPS, _RPS)])
    pltpu.sync_copy(src2d_hbm.at[pl.ds(wid * _NCH, _NCH)], sidx)
    pltpu.sync_copy(dst2d_hbm.at[pl.ds(wid * _NCH, _NCH)], didx)
    plsc.subcore_barrier()

    ebufs = (erows, rowsb)
    esems = (gsa, gsb)
    pltpu.async_copy(eattr_hbm.at[pl.ds(wid * _EPT, _C)], erows, gsa)
    for ci in range(_NCH):
        eb = ebufs[ci % 2]
        if ci + 1 < _NCH:
            pltpu.async_copy(
                eattr_hbm.at[pl.ds(wid * _EPT + (ci + 1) * _C, _C)],
                ebufs[(ci + 1) % 2], esems[(ci + 1) % 2])
        rows = rowsa
        pltpu.sync_copy(htab.at[sidx.at[ci]], rows)
        pltpu.make_async_copy(
            eattr_hbm.at[pl.ds(wid * _EPT + ci * _C, _C)], eb,
            esems[ci % 2]).wait()
        tvv = tv[...]

        def row(r, __):
            for k in range(EMB // 16):
                a = rows[r, pl.ds(k * 16, 16)]
                e = eb[r, pl.ds(k * 16, 16)]
                m = jnp.maximum(a + e, 0.0) + 1e-7
                p = jnp.exp(m * tvv)
                prows[r, pl.ds(k * 16, 16)] = p
                mrows[r, pl.ds(k * 16, 16)] = m * p
            return 0

        lax.fori_loop(0, _C, row, 0)
        pltpu.sync_copy(prows, accp.at[didx.at[ci]], add=True)
        pltpu.sync_copy(mrows, accm.at[didx.at[ci]], add=True)
    plsc.subcore_barrier()
    pltpu.sync_copy(accp.at[pl.ds(s * _RPS, _RPS)],
                    outp_hbm.at[c, pl.ds(s * _RPS, _RPS)])
    pltpu.sync_copy(accm.at[pl.ds(s * _RPS, _RPS)],
                    outm_hbm.at[c, pl.ds(s * _RPS, _RPS)])


@functools.partial(
    pl.kernel, mesh=_mesh(),
    out_type=jax.ShapeDtypeStruct((_NC, N, EMB), jnp.float32),
    scratch_types=[
        pltpu.VMEM((_NCH, _C), jnp.int32),
        pltpu.VMEM((_NCH, _C), jnp.int32),
        pltpu.VMEM((_C, EMB), jnp.float32),
        pltpu.VMEM((_C, EMB), jnp.float32),
        pltpu.VMEM((_RPS, EMB), jnp.float32),
        pltpu.VMEM_SHARED((N, EMB), jnp.float32),
        pltpu.VMEM_SHARED((N, EMB), jnp.float32),
        pltpu.SemaphoreType.DMA,
        pltpu.SemaphoreType.DMA,
    ])
def _hop_sc(z_hbm, src2d_hbm, dst2d_hbm, out_hbm, sidx, didx, rowsa, rowsb,
            zbuf, ztab, acc, gsa, gsb):
    c = lax.axis_index("c")
    s = lax.axis_index("s")
    wid = s * _NC + c
    _zero_fill(zbuf, _RPS, EMB)
    pltpu.sync_copy(zbuf, acc.at[pl.ds(s * _RPS, _RPS)])
    pltpu.sync_copy(z_hbm.at[pl.ds(s * _RPS, _RPS)],
                    ztab.at[pl.ds(s * _RPS, _RPS)])
    pltpu.sync_copy(src2d_hbm.at[pl.ds(wid * _NCH, _NCH)], sidx)
    pltpu.sync_copy(dst2d_hbm.at[pl.ds(wid * _NCH, _NCH)], didx)
    plsc.subcore_barrier()

    for ci in range(_NCH):
        pltpu.sync_copy(ztab.at[sidx.at[ci]], rowsa)
        pltpu.sync_copy(rowsa, acc.at[didx.at[ci]], add=True)
    plsc.subcore_barrier()
    pltpu.sync_copy(acc.at[pl.ds(s * _RPS, _RPS)],
                    out_hbm.at[c, pl.ds(s * _RPS, _RPS)])


def _pair_body(xi_ref, xj_ref, bi_ref, bj_ref, w_ref, fb_ref,
               mdp_ref, mask_ref, cnt_ref):
    i = pl.program_id(0)
    j = pl.program_id(1)

    @pl.when((i == 0) & (j == 0))
    def _():
        cnt_ref[...] = jnp.zeros((1, 1), jnp.float32)

    bi = bi_ref[...]
    bj = bj_ref[...]
    overlap = (jnp.min(bi) <= jnp.max(bj)) & (jnp.min(bj) <= jnp.max(bi))

    @pl.when(overlap)
    def _():
        xi = xi_ref[...]
        xj = xj_ref[...]
        m = jnp.maximum(xi[:, None, :], xj[None, :, :])
        dp = jax.lax.dot_general(
            m.reshape(_TI * _TJ, EMB), w_ref[...],
            (((1,), (0,)), ((), ())),
            preferred_element_type=jnp.float32).reshape(_TI, _TJ)
        dp = dp + fb_ref[0, 0]
        same = (bi == bj).astype(jnp.float32)
        ri = jax.lax.broadcasted_iota(jnp.int32, (_TI, _TJ), 0) + i * _TI
        ci = jax.lax.broadcasted_iota(jnp.int32, (_TI, _TJ), 1) + j * _TJ
        msk = same * (1.0 - (ri == ci).astype(jnp.float32))
        mdp_ref[...] = jax.nn.relu(dp * msk)
        mask_ref[...] = msk
        cnt_ref[...] += jnp.sum(msk).reshape(1, 1)

    @pl.when(jnp.logical_not(overlap))
    def _():
        mdp_ref[...] = jnp.zeros((_TI, _TJ), jnp.float32)
        mask_ref[...] = jnp.zeros((_TI, _TJ), jnp.float32)


def _pairwise_head(xs, batch, fc_w, fc_b):
    batchf = batch.astype(jnp.float32)
    bi = batchf.reshape(N, 1)
    bj = batchf.reshape(1, N)
    fb = fc_b.reshape(1, 1)
    grid = (N // _TI, N // _TJ)
    mdp, mask, cnt = pl.pallas_call(
        _pair_body,
        grid=grid,
        in_specs=[
            pl.BlockSpec((_TI, EMB), lambda i, j: (i, 0)),
            pl.BlockSpec((_TJ, EMB), lambda i, j: (j, 0)),
            pl.BlockSpec((_TI, 1), lambda i, j: (i, 0)),
            pl.BlockSpec((1, _TJ), lambda i, j: (0, j)),
            pl.BlockSpec((EMB, 1), lambda i, j: (0, 0)),
            pl.BlockSpec((1, 1), lambda i, j: (0, 0)),
        ],
        out_specs=[
            pl.BlockSpec((_TI, _TJ), lambda i, j: (i, j)),
            pl.BlockSpec((_TI, _TJ), lambda i, j: (i, j)),
            pl.BlockSpec((1, 1), lambda i, j: (0, 0)),
        ],
        out_shape=[
            jax.ShapeDtypeStruct((N, N), jnp.float32),
            jax.ShapeDtypeStruct((N, N), jnp.float32),
            jax.ShapeDtypeStruct((1, 1), jnp.float32),
        ],
    )(xs, xs, bi, bj, fc_w, fb)
    return mdp, mask, cnt[0, 0]


def _bn(x, g, b):
    mu = jnp.mean(x, axis=0)
    var = jnp.var(x, axis=0)
    return g * (x - mu) / jnp.sqrt(var + 1e-5) + b


def _aggr_from_parts(pp, mm):
    den = pp[0] + pp[1]
    num = mm[0] + mm[1]
    return num / (den + 1e-16)


def kernel(x, edge_index, edge_attr, batch, atom_emb, bond_emb, vn_emb,
           conv_w1, conv_b1, conv_g1, conv_be1, conv_w2, conv_b2, conv_t,
           ln_g, ln_b, vn_w1, vn_b1, vn_g1, vn_be1, vn_w2, vn_b2, vn_g2,
           vn_be2, proj_w, proj_b, fc_w, fc_b):
    src = edge_index[0]
    dst = edge_index[1]
    src2d = src.reshape(E // _C, _C)
    dst2d = dst.reshape(E // _C, _C)

    ea = edge_attr.astype(jnp.float32)
    iota8 = jnp.arange(8, dtype=jnp.float32)
    eattr = jnp.zeros((E, EMB), jnp.float32)
    for k in range(3):
        oh = (ea[:, k:k + 1] == iota8[None, :]).astype(jnp.float32)
        eattr = eattr + oh @ bond_emb[k]

    xf = x.astype(jnp.float32)
    iota64 = jnp.arange(64, dtype=jnp.float32)
    h = jnp.zeros((N, EMB), jnp.float32)
    for k in range(9):
        oh = (xf[:, k:k + 1] == iota64[None, :]).astype(jnp.float32)
        h = h + oh @ atom_emb[k]

    ohb = (batch.astype(jnp.float32)[:, None]
           == jnp.arange(NUM_GRAPHS, dtype=jnp.float32)[None, :]
           ).astype(jnp.float32)

    vn = jnp.broadcast_to(vn_emb, (NUM_GRAPHS, EMB))
    h = h + ohb @ vn

    def genconv(hh, i):
        t16 = jnp.full((16,), conv_t[i], jnp.float32)
        pp, mm = _genconv_sc(hh, eattr, src2d, dst2d, t16)
        aggr = _aggr_from_parts(pp, mm)
        out = hh + aggr
        out = jax.nn.relu(_bn(out @ conv_w1[i] + conv_b1[i],
                              conv_g1[i], conv_be1[i]))
        return out @ conv_w2[i] + conv_b2[i]

    h = genconv(h, 0)
    for i in range(1, L):
        hh = jax.nn.relu(_bn(h, ln_g[i], ln_b[i]))
        h = h + genconv(hh, i)
        j = i - 1
        vtmp = ohb.T @ h + vn
        v = jax.nn.relu(_bn(vtmp @ vn_w1[j] + vn_b1[j], vn_g1[j], vn_be1[j]))
        vn = jax.nn.relu(_bn(v @ vn_w2[j] + vn_b2[j], vn_g2[j], vn_be2[j]))
        h = h + ohb @ vn
    h = jax.nn.relu(_bn(h, ln_g[0], ln_b[0]))
    h = h + ohb @ vn

    onesm = jnp.full((N, EMB), 1.0, jnp.float32) * jnp.where(
        jnp.isnan(h[0, 0]), 2.0, 1.0)
    degp = _hop_sc(onesm, src2d, dst2d)
    deg = degp[0, :, 0] + degp[1, :, 0] + 1.0
    dis = deg ** -0.5
    xk = h
    z = dis[:, None] * xk
    preds = [xk]
    for _ in range(K_DAGNN):
        parts = _hop_sc(z, src2d, dst2d)
        xk = dis[:, None] * (parts[0] + parts[1] + z)
        z = dis[:, None] * xk
        preds.append(xk)
    scores = [jax.nn.sigmoid(p @ proj_w + proj_b[0]) for p in preds]
    xs = jnp.zeros((N, EMB), jnp.float32)
    for rs, p in zip(scores, preds):
        xs = xs + rs * p

    mdp, mask, count = _pairwise_head(xs, batch, fc_w, fc_b)
    return (mdp, mask, count)
